# baseline (speedup 1.0000x reference)
import jax
import jax.numpy as jnp
from jax import lax
from jax.experimental import pallas as pl
from jax.experimental.pallas import tpu as pltpu
from jax.experimental.pallas import tpu_sc as plsc

B, D, H, W, M = 32, 2, 128, 128, 128
HW = H * W
L = 16
B_PER = 2
CH = M // L
NROW = B_PER * D


def _body(flat_hbm, ind_hbm, mask_hbm, targ_hbm, out_hbm,
          ind_v, mask_v, idx_v, pred_v, part_v, zero_i, acc_v, out_v,
          part_s, sem, isem, msem):
    c = lax.axis_index("c")
    s = lax.axis_index("s")
    iota = lax.iota(jnp.int32, L)

    @pl.when(c == 0)
    def _work():
        w = s
        b0 = w * B_PER
        ind_cp = pltpu.async_copy(
            ind_hbm.at[pl.ds(b0 * M, B_PER * M)], ind_v, isem)
        mask_cp = pltpu.async_copy(
            mask_hbm.at[pl.ds(b0 * M, B_PER * M)], mask_v, msem)
        zero_i[...] = jnp.zeros((L,), jnp.int32)

        @pl.when(w == 0)
        def _init():
            part_v[0, pl.ds(0, L)] = jnp.zeros((L,), jnp.float32)
            part_v[0, pl.ds(L, L)] = jnp.zeros((L,), jnp.float32)
            part_v[0, pl.ds(2 * L, L)] = jnp.zeros((L,), jnp.float32)
            pltpu.sync_copy(part_v, part_s)

        for j in range(B_PER):
            for d in range(D):
                tbase = (b0 + j) * (M * D) + d
                for cc in range(CH):
                    tv = D * iota + (tbase + D * cc * L)
                    idx_v[NROW + j * D + d, pl.ds(cc * L, L)] = tv
        tcps = [
            pltpu.async_copy(targ_hbm.at[idx_v.at[NROW + k]],
                             pred_v.at[NROW + k], sem)
            for k in range(NROW)
        ]

        ind_cp.wait()
        for j in range(B_PER):
            for d in range(D):
                base = (b0 + j) * (D * HW) + d * HW
                for cc in range(CH):
                    iv = ind_v[pl.ds(j * M + cc * L, L)] + base
                    idx_v[j * D + d, pl.ds(cc * L, L)] = iv
        pcps = [
            pltpu.async_copy(flat_hbm.at[idx_v.at[k]], pred_v.at[k], sem)
            for k in range(NROW)
        ]

        mask_cp.wait()
        msum = jnp.zeros((L,), jnp.float32)
        mv = []
        for j in range(B_PER):
            for cc in range(CH):
                m = mask_v[pl.ds(j * M + cc * L, L)].astype(jnp.float32)
                mv.append(m)
                msum = msum + m
        for cp in tcps + pcps:
            cp.wait()

        acc = [jnp.zeros((L,), jnp.float32) for _ in range(D)]
        for j in range(B_PER):
            for cc in range(CH):
                m = mv[j * CH + cc]
                for d in range(D):
                    p = pred_v[j * D + d, pl.ds(cc * L, L)]
                    tg = pred_v[NROW + j * D + d, pl.ds(cc * L, L)]
                    acc[d] = acc[d] + m * jnp.abs(p - tg)

        plsc.subcore_barrier()
        part_v[0, pl.ds(0, L)] = acc[0]
        part_v[0, pl.ds(L, L)] = acc[1]
        part_v[0, pl.ds(2 * L, L)] = msum
        pltpu.sync_copy(part_v, part_s.at[zero_i.at[pl.ds(0, 1)]], add=True)
        plsc.subcore_barrier()

        @pl.when(w == 0)
        def _final():
            pltpu.sync_copy(part_s.at[0], acc_v)
            a0 = acc_v[pl.ds(0, L)]
            a1 = acc_v[pl.ds(L, L)]
            am = acc_v[pl.ds(2 * L, L)]
            t0 = jnp.float32(0.0)
            t1 = jnp.float32(0.0)
            tm = jnp.float32(0.0)
            for i in range(L):
                t0 = t0 + a0[i]
                t1 = t1 + a1[i]
                tm = tm + am[i]
            denom = jnp.full((L,), tm + jnp.float32(0.0001), jnp.float32)
            num = jnp.where(iota == 0, t0,
                            jnp.where(iota == 1, t1, jnp.float32(0.0)))
            out_v[...] = num / denom
            pltpu.sync_copy(out_v.at[pl.ds(0, D)], out_hbm)


@jax.jit
def _reg_loss_sc(flat, ind, mask, targ):
    mesh = plsc.VectorSubcoreMesh(
        core_axis_name="c", subcore_axis_name="s", num_cores=2,
        num_subcores=16)
    f = pl.kernel(
        _body,
        out_type=jax.ShapeDtypeStruct((D,), jnp.float32),
        mesh=mesh,
        scratch_types=[
            pltpu.VMEM((B_PER * M,), jnp.int32),
            pltpu.VMEM((B_PER * M,), jnp.int32),
            pltpu.VMEM((2 * NROW, M), jnp.int32),
            pltpu.VMEM((2 * NROW, M), jnp.float32),
            pltpu.VMEM((1, 3 * L), jnp.float32),
            pltpu.VMEM((L,), jnp.int32),
            pltpu.VMEM((3 * L,), jnp.float32),
            pltpu.VMEM((L,), jnp.float32),
            pltpu.VMEM_SHARED((1, 3 * L), jnp.float32),
            pltpu.SemaphoreType.DMA,
            pltpu.SemaphoreType.DMA,
            pltpu.SemaphoreType.DMA,
        ],
    )
    return f(flat, ind, mask, targ)


def kernel(output, mask, ind, target):
    flat = output.reshape(-1)
    ind32 = ind.reshape(-1).astype(jnp.int32)
    mask32 = mask.reshape(-1).astype(jnp.int32)
    targf = target.reshape(-1)
    return _reg_loss_sc(flat, ind32, mask32, targf)


# speedup vs baseline: 1.0002x; 1.0002x over previous
import jax
import jax.numpy as jnp
from jax import lax
from jax.experimental import pallas as pl
from jax.experimental.pallas import tpu as pltpu
from jax.experimental.pallas import tpu_sc as plsc

B, D, H, W, M = 32, 2, 128, 128, 128
HW = H * W
L = 16
B_PER = 2
CH = M // L
NROW = B_PER * D


def _body(flat_hbm, ind_hbm, mask_hbm, targ_hbm, out_hbm,
          ind_v, mask_v, pidx_v, tidx_v, ppred_v, tpred_v, part_v, zero_i,
          acc_v, out_v, part_s, sem, isem, msem):
    c = lax.axis_index("c")
    s = lax.axis_index("s")
    iota = lax.iota(jnp.int32, L)

    @pl.when(c == 0)
    def _work():
        w = s
        b0 = w * B_PER
        ind_cp = pltpu.async_copy(
            ind_hbm.at[pl.ds(b0 * M, B_PER * M)], ind_v, isem)
        mask_cp = pltpu.async_copy(
            mask_hbm.at[pl.ds(b0 * M, B_PER * M)], mask_v, msem)
        zero_i[...] = jnp.zeros((L,), jnp.int32)

        @pl.when(w == 0)
        def _init():
            part_v[0, pl.ds(0, L)] = jnp.zeros((L,), jnp.float32)
            part_v[0, pl.ds(L, L)] = jnp.zeros((L,), jnp.float32)
            part_v[0, pl.ds(2 * L, L)] = jnp.zeros((L,), jnp.float32)
            pltpu.sync_copy(part_v, part_s)

        for j in range(B_PER):
            for d in range(D):
                tbase = (b0 + j) * (M * D) + d
                for cc in range(CH):
                    tv = D * iota + (tbase + D * cc * L)
                    tidx_v[pl.ds((j * D + d) * M + cc * L, L)] = tv
        tcp = pltpu.async_copy(targ_hbm.at[tidx_v], tpred_v, sem)

        ind_cp.wait()
        for j in range(B_PER):
            for d in range(D):
                base = (b0 + j) * (D * HW) + d * HW
                for cc in range(CH):
                    iv = ind_v[pl.ds(j * M + cc * L, L)] + base
                    pidx_v[pl.ds((j * D + d) * M + cc * L, L)] = iv
        pcp = pltpu.async_copy(flat_hbm.at[pidx_v], ppred_v, sem)

        mask_cp.wait()
        msum = jnp.zeros((L,), jnp.float32)
        mv = []
        for j in range(B_PER):
            for cc in range(CH):
                m = mask_v[pl.ds(j * M + cc * L, L)].astype(jnp.float32)
                mv.append(m)
                msum = msum + m
        tcp.wait()
        pcp.wait()

        acc = [jnp.zeros((L,), jnp.float32) for _ in range(D)]
        for j in range(B_PER):
            for cc in range(CH):
                m = mv[j * CH + cc]
                for d in range(D):
                    p = ppred_v[pl.ds((j * D + d) * M + cc * L, L)]
                    tg = tpred_v[pl.ds((j * D + d) * M + cc * L, L)]
                    acc[d] = acc[d] + m * jnp.abs(p - tg)

        plsc.subcore_barrier()
        part_v[0, pl.ds(0, L)] = acc[0]
        part_v[0, pl.ds(L, L)] = acc[1]
        part_v[0, pl.ds(2 * L, L)] = msum
        pltpu.sync_copy(part_v, part_s.at[zero_i.at[pl.ds(0, 1)]], add=True)
        plsc.subcore_barrier()

        @pl.when(w == 0)
        def _final():
            pltpu.sync_copy(part_s.at[0], acc_v)
            a0 = acc_v[pl.ds(0, L)]
            a1 = acc_v[pl.ds(L, L)]
            am = acc_v[pl.ds(2 * L, L)]
            t0 = jnp.float32(0.0)
            t1 = jnp.float32(0.0)
            tm = jnp.float32(0.0)
            for i in range(L):
                t0 = t0 + a0[i]
                t1 = t1 + a1[i]
                tm = tm + am[i]
            denom = jnp.full((L,), tm + jnp.float32(0.0001), jnp.float32)
            num = jnp.where(iota == 0, t0,
                            jnp.where(iota == 1, t1, jnp.float32(0.0)))
            out_v[...] = num / denom
            pltpu.sync_copy(out_v.at[pl.ds(0, D)], out_hbm)


@jax.jit
def _reg_loss_sc(flat, ind, mask, targ):
    mesh = plsc.VectorSubcoreMesh(
        core_axis_name="c", subcore_axis_name="s", num_cores=2,
        num_subcores=16)
    f = pl.kernel(
        _body,
        out_type=jax.ShapeDtypeStruct((D,), jnp.float32),
        mesh=mesh,
        scratch_types=[
            pltpu.VMEM((B_PER * M,), jnp.int32),
            pltpu.VMEM((B_PER * M,), jnp.int32),
            pltpu.VMEM((NROW * M,), jnp.int32),
            pltpu.VMEM((NROW * M,), jnp.int32),
            pltpu.VMEM((NROW * M,), jnp.float32),
            pltpu.VMEM((NROW * M,), jnp.float32),
            pltpu.VMEM((1, 3 * L), jnp.float32),
            pltpu.VMEM((L,), jnp.int32),
            pltpu.VMEM((3 * L,), jnp.float32),
            pltpu.VMEM((L,), jnp.float32),
            pltpu.VMEM_SHARED((1, 3 * L), jnp.float32),
            pltpu.SemaphoreType.DMA,
            pltpu.SemaphoreType.DMA,
            pltpu.SemaphoreType.DMA,
        ],
    )
    return f(flat, ind, mask, targ)


def kernel(output, mask, ind, target):
    flat = output.reshape(-1)
    ind32 = ind.reshape(-1).astype(jnp.int32)
    mask32 = mask.reshape(-1).astype(jnp.int32)
    targf = target.reshape(-1)
    return _reg_loss_sc(flat, ind32, mask32, targf)


# speedup vs baseline: 1.0780x; 1.0778x over previous
import jax
import jax.numpy as jnp
from jax import lax
from jax.experimental import pallas as pl
from jax.experimental.pallas import tpu as pltpu
from jax.experimental.pallas import tpu_sc as plsc

B, D, H, W, M = 32, 2, 128, 128, 128
HW = H * W
L = 16
B_PER = 2
CH = M // L
NROW = B_PER * D


def _body(flat_hbm, ind_hbm, mask_hbm, targ_hbm, out_hbm,
          ind_v, mask_v, pidx_v, tidx_v, ppred_v, tpred_v, part_v, zero_i,
          acc_v, out_v, part_s, sem, isem, msem):
    c = lax.axis_index("c")
    s = lax.axis_index("s")
    iota = lax.iota(jnp.int32, L)

    @pl.when(c == 0)
    def _work():
        w = s
        b0 = w * B_PER
        ind_cp = pltpu.async_copy(
            ind_hbm.at[pl.ds(b0 * M, B_PER * M)], ind_v, isem)
        mask_cp = pltpu.async_copy(
            mask_hbm.at[pl.ds(b0 * M, B_PER * M)], mask_v, msem)
        zero_i[...] = jnp.zeros((L,), jnp.int32)

        @pl.when(w == 0)
        def _init():
            part_v[0, pl.ds(0, L)] = jnp.zeros((L,), jnp.float32)
            part_v[0, pl.ds(L, L)] = jnp.zeros((L,), jnp.float32)
            part_v[0, pl.ds(2 * L, L)] = jnp.zeros((L,), jnp.float32)
            pltpu.sync_copy(part_v, part_s)

        for j in range(B_PER):
            for d in range(D):
                tbase = (b0 + j) * (M * D) + d
                for cc in range(CH):
                    tv = D * iota + (tbase + D * cc * L)
                    tidx_v[pl.ds((j * D + d) * M + cc * L, L)] = tv
        tcp = pltpu.async_copy(targ_hbm.at[tidx_v], tpred_v, sem)

        ind_cp.wait()
        for j in range(B_PER):
            for d in range(D):
                base = (b0 + j) * (D * HW) + d * HW
                for cc in range(CH):
                    iv = ind_v[pl.ds(j * M + cc * L, L)] + base
                    pidx_v[pl.ds((j * D + d) * M + cc * L, L)] = iv
        pcp = pltpu.async_copy(flat_hbm.at[pidx_v], ppred_v, sem)

        mask_cp.wait()
        msum = jnp.zeros((L,), jnp.float32)
        mv = []
        for j in range(B_PER):
            for cc in range(CH):
                m = mask_v[pl.ds(j * M + cc * L, L)].astype(jnp.float32)
                mv.append(m)
                msum = msum + m
        tcp.wait()
        pcp.wait()

        acc = [jnp.zeros((L,), jnp.float32) for _ in range(D)]
        for j in range(B_PER):
            for cc in range(CH):
                m = mv[j * CH + cc]
                for d in range(D):
                    p = ppred_v[pl.ds((j * D + d) * M + cc * L, L)]
                    tg = tpred_v[pl.ds((j * D + d) * M + cc * L, L)]
                    acc[d] = acc[d] + m * jnp.abs(p - tg)

        plsc.subcore_barrier()
        part_v[0, pl.ds(0, L)] = acc[0]
        part_v[0, pl.ds(L, L)] = acc[1]
        part_v[0, pl.ds(2 * L, L)] = msum
        pltpu.sync_copy(part_v, part_s.at[zero_i.at[pl.ds(0, 1)]], add=True)
        plsc.subcore_barrier()

        @pl.when(w == 0)
        def _final():
            pltpu.sync_copy(part_s.at[0], acc_v)
            a0 = acc_v[pl.ds(0, L)]
            a1 = acc_v[pl.ds(L, L)]
            am = acc_v[pl.ds(2 * L, L)]
            t0 = jnp.float32(0.0)
            t1 = jnp.float32(0.0)
            tm = jnp.float32(0.0)
            for i in range(L):
                t0 = t0 + a0[i]
                t1 = t1 + a1[i]
                tm = tm + am[i]
            denom = jnp.full((L,), tm + jnp.float32(0.0001), jnp.float32)
            num = jnp.where(iota == 0, t0,
                            jnp.where(iota == 1, t1, jnp.float32(0.0)))
            out_v[...] = num / denom
            pltpu.sync_copy(out_v.at[pl.ds(0, D)], out_hbm)


@jax.jit
def _reg_loss_sc(flat, ind, mask, targ):
    mesh = plsc.VectorSubcoreMesh(
        core_axis_name="c", subcore_axis_name="s", num_cores=1,
        num_subcores=16)
    f = pl.kernel(
        _body,
        out_type=jax.ShapeDtypeStruct((D,), jnp.float32),
        mesh=mesh,
        scratch_types=[
            pltpu.VMEM((B_PER * M,), jnp.int32),
            pltpu.VMEM((B_PER * M,), jnp.int32),
            pltpu.VMEM((NROW * M,), jnp.int32),
            pltpu.VMEM((NROW * M,), jnp.int32),
            pltpu.VMEM((NROW * M,), jnp.float32),
            pltpu.VMEM((NROW * M,), jnp.float32),
            pltpu.VMEM((1, 3 * L), jnp.float32),
            pltpu.VMEM((L,), jnp.int32),
            pltpu.VMEM((3 * L,), jnp.float32),
            pltpu.VMEM((L,), jnp.float32),
            pltpu.VMEM_SHARED((1, 3 * L), jnp.float32),
            pltpu.SemaphoreType.DMA,
            pltpu.SemaphoreType.DMA,
            pltpu.SemaphoreType.DMA,
        ],
    )
    return f(flat, ind, mask, targ)


def kernel(output, mask, ind, target):
    flat = output.reshape(-1)
    ind32 = ind.reshape(-1).astype(jnp.int32)
    mask32 = mask.reshape(-1).astype(jnp.int32)
    targf = target.reshape(-1)
    return _reg_loss_sc(flat, ind32, mask32, targf)
